# baseline (speedup 1.0000x reference)
import jax
import jax.numpy as jnp
from jax.experimental import pallas as pl
from jax.experimental.pallas import tpu as pltpu
from jax.experimental.pallas import tpu_sc as plsc

D_MODEL = 4096
NUM_EXPERTS = 64
TOP_K = 8

TI = 512
NJ = 2
DJ = D_MODEL // NJ
N_TOKENS = 4096
NI = N_TOKENS // TI

N_TILES = 32
RPT = N_TOKENS // N_TILES

_CONTRACT_LAST = (((1,), (1,)), ((), ()))


def _router_tc_kernel(temp_ref, x_ref, w_ref, sim_ref, scores_ref,
                      s_acc, n2_acc, simn_acc):
    j = pl.program_id(0)
    i = pl.program_id(1)

    p = jax.lax.dot_general(x_ref[...], w_ref[...], _CONTRACT_LAST,
                            preferred_element_type=jnp.float32)
    n2_part = jnp.sum(p * p, axis=1, keepdims=True)
    s_part = jax.lax.dot_general(p, sim_ref[...], _CONTRACT_LAST,
                                 preferred_element_type=jnp.float32)

    rows = pl.ds(i * TI, TI)

    @pl.when(j == 0)
    def _init():
        s_acc[rows, :] = s_part
        n2_acc[rows, :] = n2_part

    @pl.when(j > 0)
    def _accum():
        s_acc[rows, :] = s_acc[rows, :] + s_part
        n2_acc[rows, :] = n2_acc[rows, :] + n2_part

    @pl.when(i == 0)
    def _simnorm():
        ssq = sim_ref[...] * sim_ref[...]
        part = jax.lax.dot_general(
            jnp.ones((1, DJ), jnp.float32), ssq, _CONTRACT_LAST,
            preferred_element_type=jnp.float32)

        @pl.when(j == 0)
        def _():
            simn_acc[...] = part

        @pl.when(j > 0)
        def _():
            simn_acc[...] = simn_acc[...] + part

        @pl.when(j == NJ - 1)
        def _():
            temp = temp_ref[0, 0]
            simn_acc[...] = 1.0 / (
                jnp.maximum(jnp.sqrt(simn_acc[...]), 1e-12) * temp)

    @pl.when(j == NJ - 1)
    def _finalize():
        s = s_acc[rows, :]
        pnorm = jnp.maximum(jnp.sqrt(n2_acc[rows, :]), 1e-12)
        scores_ref[...] = (s / pnorm) * simn_acc[...]


_GDN = jax.lax.GatherDimensionNumbers(
    offset_dims=(), collapsed_slice_dims=(0,), start_index_map=(0,))


def _perm(v, idx):
    return jax.lax.gather(v, idx[:, None], _GDN, (1,),
                          mode=jax.lax.GatherScatterMode.PROMISE_IN_BOUNDS)


def _sc_tail_body(scores_hbm, probs_hbm, usage_hbm, psum_hbm,
                  scores_vmem, probs_vmem, usage_vmem, psum_vmem, tmp_vmem,
                  sem_in, sem_out, sem_u, sem_p):
    c = jax.lax.axis_index("c")
    s = jax.lax.axis_index("s")
    tid = c * 16 + s
    row0 = tid * RPT

    cp_in = pltpu.make_async_copy(
        scores_hbm.at[pl.ds(row0, RPT), :], scores_vmem, sem_in)
    cp_in.start()
    cp_in.wait()

    lane = jax.lax.iota(jnp.int32, 16)
    lane_f = lane.astype(jnp.float32)
    gids = [lane_f + (16.0 * q) for q in range(4)]
    NEG = jnp.float32(-jnp.inf)

    def _bfly(v, op):
        for sh in (1, 2, 4, 8):
            v = op(v, _perm(v, lane ^ sh))
        return v

    def row_body(r, carry):
        u0, u1, u2, u3, msum = carry
        us = [u0, u1, u2, u3]
        w = [scores_vmem[r, pl.ds(16 * q, 16)] for q in range(4)]
        vvals = jnp.zeros((16,), jnp.float32)
        val0 = jnp.zeros((16,), jnp.float32)
        for k in range(TOP_K):
            m4 = jnp.maximum(jnp.maximum(w[0], w[1]),
                             jnp.maximum(w[2], w[3]))
            mb = _bfly(m4, jnp.maximum)
            eqs = [wq == mb for wq in w]
            ids = [jnp.where(eqs[q], gids[q], 64.0) for q in range(4)]
            i4 = jnp.minimum(jnp.minimum(ids[0], ids[1]),
                             jnp.minimum(ids[2], ids[3]))
            ib = _bfly(i4, jnp.minimum)
            new_w, new_u = [], []
            for q in range(4):
                sel = jnp.logical_and(eqs[q], gids[q] == ib)
                new_w.append(jnp.where(sel, NEG, w[q]))
                new_u.append(us[q] + jnp.where(sel, 1.0, 0.0))
            w, us = new_w, new_u
            vvals = jnp.where(lane == k, mb, vvals)
            if k == 0:
                val0 = mb
        ex = jnp.exp(vvals - val0)
        ex = jnp.where(lane < TOP_K, ex, 0.0)
        probs = ex / _bfly(ex, jnp.add)
        probs_vmem[r, :] = probs
        return (us[0], us[1], us[2], us[3], msum + probs)

    z = jnp.zeros((16,), jnp.float32)
    u0, u1, u2, u3, msum = jax.lax.fori_loop(
        0, RPT, row_body, (z, z, z, z, z))

    for q, u in enumerate([u0, u1, u2, u3]):
        usage_vmem[pl.ds(16 * q, 16)] = u
    psum_vmem[...] = msum

    cp_out = pltpu.make_async_copy(
        probs_vmem, probs_hbm.at[pl.ds(row0, RPT), :], sem_out)
    cp_out.start()
    cp_u = pltpu.make_async_copy(usage_vmem, usage_hbm.at[tid, :], sem_u)
    cp_u.start()
    cp_p = pltpu.make_async_copy(psum_vmem, psum_hbm.at[tid, :], sem_p)
    cp_p.start()
    cp_out.wait()
    cp_u.wait()
    cp_p.wait()


def kernel(x, W, sim_matrix, temperature):
    B, T, _ = x.shape
    x2d = x.reshape(N_TOKENS, D_MODEL)
    temp = jnp.asarray(temperature, jnp.float32).reshape(1, 1)

    scores = pl.pallas_call(
        _router_tc_kernel,
        grid=(NJ, NI),
        in_specs=[
            pl.BlockSpec(memory_space=pltpu.SMEM),
            pl.BlockSpec((TI, D_MODEL), lambda j, i: (i, 0)),
            pl.BlockSpec((DJ, D_MODEL), lambda j, i: (j, 0),
                         pipeline_mode=pl.Buffered(buffer_count=1)),
            pl.BlockSpec((NUM_EXPERTS, DJ), lambda j, i: (0, j)),
        ],
        out_specs=pl.BlockSpec((TI, NUM_EXPERTS), lambda j, i: (i, 0)),
        out_shape=jax.ShapeDtypeStruct((N_TOKENS, NUM_EXPERTS), jnp.float32),
        scratch_shapes=[
            pltpu.VMEM((N_TOKENS, NUM_EXPERTS), jnp.float32),
            pltpu.VMEM((N_TOKENS, 1), jnp.float32),
            pltpu.VMEM((1, NUM_EXPERTS), jnp.float32),
        ],
        compiler_params=pltpu.CompilerParams(
            dimension_semantics=("arbitrary", "arbitrary"),
        ),
    )(temp, x2d, W, sim_matrix)

    probs_pad, usage_parts, psum_parts = pl.kernel(
        _sc_tail_body,
        out_type=[
            jax.ShapeDtypeStruct((N_TOKENS, 16), jnp.float32),
            jax.ShapeDtypeStruct((N_TILES, NUM_EXPERTS), jnp.float32),
            jax.ShapeDtypeStruct((N_TILES, 16), jnp.float32),
        ],
        mesh=plsc.VectorSubcoreMesh(core_axis_name="c", subcore_axis_name="s"),
        scratch_types=[
            pltpu.VMEM((RPT, NUM_EXPERTS), jnp.float32),
            pltpu.VMEM((RPT, 16), jnp.float32),
            pltpu.VMEM((NUM_EXPERTS,), jnp.float32),
            pltpu.VMEM((16,), jnp.float32),
            pltpu.VMEM((16,), jnp.float32),
            pltpu.SemaphoreType.DMA,
            pltpu.SemaphoreType.DMA,
            pltpu.SemaphoreType.DMA,
            pltpu.SemaphoreType.DMA,
        ],
    )(scores)

    gate_scores = scores.reshape(B, T, NUM_EXPERTS)
    gate_probs = probs_pad[:, :TOP_K].reshape(B, T, TOP_K)
    expert_usage = jnp.sum(usage_parts, axis=0)
    gate_probs_mean = jnp.sum(psum_parts) * (1.0 / (N_TOKENS * TOP_K))
    return gate_scores, gate_probs, expert_usage, gate_probs_mean
